# baseline (speedup 1.0000x reference)
import functools

import jax
import jax.numpy as jnp
from jax import lax
from jax.experimental import pallas as pl
from jax.experimental.pallas import tpu as pltpu
from jax.experimental.pallas import tpu_sc as plsc

B = 16384
E = 32
NC = 2
NS = 16
L = 16
NW = NC * NS
BPW = B // NW
CHUNK = 128
NCHUNK = BPW // CHUNK
NGROUP = BPW // L


def _sc_body(uidx_hbm, bidx_hbm, utab_hbm, btab_hbm, ubias_hbm, bbias_hbm,
             out_hbm, uidx_v, bidx_v, urows_v, brows_v, ub_v, bb_v, out_v,
             sem):
    wid = lax.axis_index("s") * NC + lax.axis_index("c")
    base = wid * BPW

    pltpu.sync_copy(uidx_hbm.at[pl.ds(base, BPW)], uidx_v)
    pltpu.sync_copy(bidx_hbm.at[pl.ds(base, BPW)], bidx_v)

    copies = []
    for c in range(NCHUNK):
        sl = pl.ds(c * CHUNK, CHUNK)
        copies.append(pltpu.async_copy(utab_hbm.at[uidx_v.at[sl]],
                                       urows_v.at[sl], sem))
        copies.append(pltpu.async_copy(btab_hbm.at[bidx_v.at[sl]],
                                       brows_v.at[sl], sem))
        copies.append(pltpu.async_copy(ubias_hbm.at[uidx_v.at[sl]],
                                       ub_v.at[sl], sem))
        copies.append(pltpu.async_copy(bbias_hbm.at[bidx_v.at[sl]],
                                       bb_v.at[sl], sem))
    for cp in copies:
        cp.wait()

    lane = lax.iota(jnp.int32, L)

    def group(g, carry):
        pbase = g * L
        pidx = pbase + lane
        acc = jnp.zeros((L,), jnp.float32)
        for e in range(E):
            eidx = jnp.full((L,), e, jnp.int32)
            u = plsc.load_gather(urows_v, [pidx, eidx])
            b = plsc.load_gather(brows_v, [pidx, eidx])
            acc = acc + u * b
        x = acc + ub_v[pl.ds(pbase, L)] + bb_v[pl.ds(pbase, L)]
        out_v[pl.ds(pbase, L)] = 1.0 / (1.0 + jnp.exp(-x))
        return carry

    lax.fori_loop(0, NGROUP, group, 0)

    pltpu.sync_copy(out_v, out_hbm.at[pl.ds(base, BPW)])


_sc_call = pl.kernel(
    _sc_body,
    out_type=jax.ShapeDtypeStruct((B,), jnp.float32),
    mesh=plsc.VectorSubcoreMesh(core_axis_name="c", subcore_axis_name="s"),
    compiler_params=pltpu.CompilerParams(needs_layout_passes=False,
                                         use_tc_tiling_on_sc=False),
    scratch_types=[
        pltpu.VMEM((BPW,), jnp.int32),
        pltpu.VMEM((BPW,), jnp.int32),
        pltpu.VMEM((BPW, E), jnp.float32),
        pltpu.VMEM((BPW, E), jnp.float32),
        pltpu.VMEM((BPW,), jnp.float32),
        pltpu.VMEM((BPW,), jnp.float32),
        pltpu.VMEM((BPW,), jnp.float32),
        pltpu.SemaphoreType.DMA,
    ],
)


def kernel(inputs, user_embedding, user_bias, book_embedding, book_bias):
    uidx = inputs[:, 0]
    bidx = inputs[:, 1]
    out = _sc_call(uidx, bidx, user_embedding, book_embedding,
                   user_bias[:, 0], book_bias[:, 0])
    return out.reshape(B, 1)
